# baseline (speedup 1.0000x reference)
import functools

import jax
import jax.numpy as jnp
from jax import lax
from jax.experimental import pallas as pl
from jax.experimental.pallas import tpu as pltpu
from jax.experimental.pallas import tpu_sc as plsc

NC = 2
NS = 16
L = 16
NW = NC * NS

B = 16384
D = 64
NE = 1000000
NP = 1000
BPW = B // NW
CHUNK = 128
NCHUNK = BPW // CHUNK
NG = BPW // L
RING = 2

_EPS = 1e-8


def _sqrt16(x):
    xg = jnp.maximum(x, 1e-30)
    i = plsc.bitcast(xg, jnp.int32)
    y = plsc.bitcast(jnp.full((L,), 0x5F3759DF, jnp.int32) - (i >> 1),
                     jnp.float32)
    for _ in range(3):
        y = y * (1.5 - 0.5 * xg * y * y)
    return x * y


def _body(pred_ids_hbm, ent_ids_hbm, ent_flat_hbm, pred_flat_hbm, out_hbm,
          eidx_v, pidx_v, ring_v, evals_v, ptab_v, out_v,
          sem_ids, sem_ptab, sem_g):
    wid = lax.axis_index("s") * NC + lax.axis_index("c")
    base = wid * BPW

    c1 = pltpu.async_copy(ent_ids_hbm.at[pl.ds(base, BPW)], eidx_v, sem_ids)
    c2 = pltpu.async_copy(pred_ids_hbm.at[pl.ds(base, BPW)], pidx_v, sem_ids)
    cp = pltpu.async_copy(pred_flat_hbm.at[pl.ds(0, D * NP)], ptab_v, sem_ptab)
    c1.wait()
    c2.wait()

    def fire(d, slot):
        for j in range(NG):
            sl = pl.ds(j * L, L)
            ring_v[slot, sl] = eidx_v[sl] + d * NE
        handles = []
        for j in range(NCHUNK):
            handles.append(pltpu.async_copy(
                ent_flat_hbm.at[ring_v.at[slot, pl.ds(j * CHUNK, CHUNK)]],
                evals_v.at[pl.ds(d * BPW + j * CHUNK, CHUNK)], sem_g))
        return handles

    def drain(handles):
        for h in handles:
            h.wait()

    inflight = []
    for d in range(RING):
        inflight.append(fire(d, d))
    for d in range(RING, D):
        drain(inflight[d - RING])
        inflight.append(fire(d, d % RING))
    for d in range(D - RING, D):
        drain(inflight[d])

    cp.wait()

    def group(g, carry):
        pids = pidx_v[pl.ds(g * L, L)]
        num = jnp.zeros((L,), jnp.float32)
        e2 = jnp.zeros((L,), jnp.float32)
        p2 = jnp.zeros((L,), jnp.float32)
        for dd in range(D):
            e = evals_v[pl.ds(dd * BPW + g * L, L)]
            p = plsc.load_gather(ptab_v, [pids + dd * NP])
            num = num + e * p
            e2 = e2 + e * e
            p2 = p2 + p * p
        denom = jnp.maximum(_sqrt16(e2) * _sqrt16(p2), _EPS)
        truth = 0.5 * (num / denom) + 0.5
        out_v[pl.ds(g * L, L)] = truth
        return carry

    lax.fori_loop(0, NG, group, 0)

    pltpu.sync_copy(out_v, out_hbm.at[pl.ds(base, BPW)])


_mesh = plsc.VectorSubcoreMesh(core_axis_name="c", subcore_axis_name="s",
                               num_cores=NC, num_subcores=NS)

_sc_call = pl.kernel(
    _body,
    out_type=jax.ShapeDtypeStruct((B,), jnp.float32),
    mesh=_mesh,
    scratch_types=[
        pltpu.VMEM((BPW,), jnp.int32),
        pltpu.VMEM((BPW,), jnp.int32),
        pltpu.VMEM((RING, BPW), jnp.int32),
        pltpu.VMEM((D * BPW,), jnp.float32),
        pltpu.VMEM((D * NP,), jnp.float32),
        pltpu.VMEM((BPW,), jnp.float32),
        pltpu.SemaphoreType.DMA,
        pltpu.SemaphoreType.DMA,
        pltpu.SemaphoreType.DMA,
    ],
    compiler_params=pltpu.CompilerParams(needs_layout_passes=False,
                                         use_tc_tiling_on_sc=False),
)


@jax.jit
def kernel(predicate_ids, entity_ids, entity_table, predicate_table):
    ent_flat = entity_table.T.reshape(D * NE)
    pred_flat = predicate_table.T.reshape(D * NP)
    return _sc_call(predicate_ids, entity_ids, ent_flat, pred_flat)


# speedup vs baseline: 8.6315x; 8.6315x over previous
import functools

import jax
import jax.numpy as jnp
from jax import lax
from jax.experimental import pallas as pl
from jax.experimental.pallas import tpu as pltpu
from jax.experimental.pallas import tpu_sc as plsc

NC = 2
NS = 16
L = 16
NW = NC * NS

B = 16384
D = 64
DP = 128
BPW = B // NW
CH = 128
NCH = BPW // CH
NG = CH // L

_EPS = 1e-8


def _sqrt16(x):
    xg = jnp.maximum(x, 1e-30)
    i = plsc.bitcast(xg, jnp.int32)
    y = plsc.bitcast(jnp.full((L,), 0x5F3759DF, jnp.int32) - (i >> 1),
                     jnp.float32)
    for _ in range(3):
        y = y * (1.5 - 0.5 * xg * y * y)
    return x * y


def _body(pred_ids_hbm, ent_ids_hbm, ent_tab_hbm, pred_tab_hbm, out_hbm,
          eidx_v, pidx_v, erows_v, prows_v, out_v, sem):
    wid = lax.axis_index("s") * NC + lax.axis_index("c")
    base = wid * BPW

    c1 = pltpu.async_copy(ent_ids_hbm.at[pl.ds(base, BPW)], eidx_v, sem)
    c2 = pltpu.async_copy(pred_ids_hbm.at[pl.ds(base, BPW)], pidx_v, sem)
    c1.wait()
    c2.wait()

    def chunk(ch, carry):
        cbase = ch * CH
        ge = pltpu.async_copy(
            ent_tab_hbm.at[eidx_v.at[pl.ds(cbase, CH)]], erows_v, sem)
        gp = pltpu.async_copy(
            pred_tab_hbm.at[pidx_v.at[pl.ds(cbase, CH)]], prows_v, sem)
        ge.wait()
        gp.wait()

        def group(g, carry2):
            k = g * L + lax.iota(jnp.int32, L)
            num = jnp.zeros((L,), jnp.float32)
            e2 = jnp.zeros((L,), jnp.float32)
            p2 = jnp.zeros((L,), jnp.float32)
            for dd in range(D):
                col = jnp.full((L,), dd, jnp.int32)
                e = plsc.load_gather(erows_v, [k, col])
                p = plsc.load_gather(prows_v, [k, col])
                num = num + e * p
                e2 = e2 + e * e
                p2 = p2 + p * p
            denom = jnp.maximum(_sqrt16(e2) * _sqrt16(p2), _EPS)
            truth = 0.5 * (num / denom) + 0.5
            out_v[pl.ds(cbase + g * L, L)] = truth
            return carry2

        lax.fori_loop(0, NG, group, 0)
        return carry

    lax.fori_loop(0, NCH, chunk, 0)

    pltpu.sync_copy(out_v, out_hbm.at[pl.ds(base, BPW)])


_mesh = plsc.VectorSubcoreMesh(core_axis_name="c", subcore_axis_name="s",
                               num_cores=NC, num_subcores=NS)

_sc_call = pl.kernel(
    _body,
    out_type=jax.ShapeDtypeStruct((B,), jnp.float32),
    mesh=_mesh,
    scratch_types=[
        pltpu.VMEM((BPW,), jnp.int32),
        pltpu.VMEM((BPW,), jnp.int32),
        pltpu.VMEM((CH, DP), jnp.float32),
        pltpu.VMEM((CH, DP), jnp.float32),
        pltpu.VMEM((BPW,), jnp.float32),
        pltpu.SemaphoreType.DMA,
    ],
    compiler_params=pltpu.CompilerParams(needs_layout_passes=False,
                                         use_tc_tiling_on_sc=True),
)


@jax.jit
def kernel(predicate_ids, entity_ids, entity_table, predicate_table):
    ent_pad = jnp.pad(entity_table, ((0, 0), (0, DP - D)))
    pred_pad = jnp.pad(predicate_table, ((0, 0), (0, DP - D)))
    return _sc_call(predicate_ids, entity_ids, ent_pad, pred_pad)


# speedup vs baseline: 8.7123x; 1.0094x over previous
import functools

import jax
import jax.numpy as jnp
from jax import lax
from jax.experimental import pallas as pl
from jax.experimental.pallas import tpu as pltpu
from jax.experimental.pallas import tpu_sc as plsc

NC = 2
NS = 16
L = 16
NW = NC * NS

B = 16384
D = 64
DP = 128
BPW = B // NW
CH = 128
NCH = BPW // CH
NG = CH // L

_EPS = 1e-8


def _sqrt16(x):
    xg = jnp.maximum(x, 1e-30)
    i = plsc.bitcast(xg, jnp.int32)
    y = plsc.bitcast(jnp.full((L,), 0x5F3759DF, jnp.int32) - (i >> 1),
                     jnp.float32)
    for _ in range(3):
        y = y * (1.5 - 0.5 * xg * y * y)
    return x * y


def _body(pred_ids_hbm, ent_ids_hbm, ent_tab_hbm, pred_tab_hbm, out_hbm,
          eidx_v, pidx_v, erows_v, prows_v, out_v, sem):
    wid = lax.axis_index("s") * NC + lax.axis_index("c")
    base = wid * BPW

    c1 = pltpu.async_copy(ent_ids_hbm.at[pl.ds(base, BPW)], eidx_v, sem)
    c2 = pltpu.async_copy(pred_ids_hbm.at[pl.ds(base, BPW)], pidx_v, sem)
    c1.wait()
    c2.wait()

    def fire(ch):
        cbase = ch * CH
        buf = ch & 1
        ge = pltpu.async_copy(
            ent_tab_hbm.at[eidx_v.at[pl.ds(cbase, CH)]],
            erows_v.at[buf], sem)
        gp = pltpu.async_copy(
            pred_tab_hbm.at[pidx_v.at[pl.ds(cbase, CH)]],
            prows_v.at[buf], sem)
        return ge, gp

    pending = {0: fire(0)}
    for ch in range(NCH):
        if ch + 1 < NCH:
            pending[ch + 1] = fire(ch + 1)
        ge, gp = pending.pop(ch)
        ge.wait()
        gp.wait()
        cbase = ch * CH
        ebuf = erows_v.at[ch & 1]
        pbuf = prows_v.at[ch & 1]

        def group(g, carry2, ebuf=ebuf, pbuf=pbuf, cbase=cbase):
            k = g * L + lax.iota(jnp.int32, L)
            num = jnp.zeros((L,), jnp.float32)
            e2 = jnp.zeros((L,), jnp.float32)
            p2 = jnp.zeros((L,), jnp.float32)
            for dd in range(D):
                col = jnp.full((L,), dd, jnp.int32)
                e = plsc.load_gather(ebuf, [k, col])
                p = plsc.load_gather(pbuf, [k, col])
                num = num + e * p
                e2 = e2 + e * e
                p2 = p2 + p * p
            denom = jnp.maximum(_sqrt16(e2) * _sqrt16(p2), _EPS)
            truth = 0.5 * (num / denom) + 0.5
            out_v[pl.ds(cbase + g * L, L)] = truth
            return carry2

        lax.fori_loop(0, NG, group, 0)

    pltpu.sync_copy(out_v, out_hbm.at[pl.ds(base, BPW)])


_mesh = plsc.VectorSubcoreMesh(core_axis_name="c", subcore_axis_name="s",
                               num_cores=NC, num_subcores=NS)

_sc_call = pl.kernel(
    _body,
    out_type=jax.ShapeDtypeStruct((B,), jnp.float32),
    mesh=_mesh,
    scratch_types=[
        pltpu.VMEM((BPW,), jnp.int32),
        pltpu.VMEM((BPW,), jnp.int32),
        pltpu.VMEM((2, CH, DP), jnp.float32),
        pltpu.VMEM((2, CH, DP), jnp.float32),
        pltpu.VMEM((BPW,), jnp.float32),
        pltpu.SemaphoreType.DMA,
    ],
    compiler_params=pltpu.CompilerParams(needs_layout_passes=False,
                                         use_tc_tiling_on_sc=True),
)


@jax.jit
def kernel(predicate_ids, entity_ids, entity_table, predicate_table):
    ent_pad = jnp.pad(entity_table, ((0, 0), (0, DP - D)))
    pred_pad = jnp.pad(predicate_table, ((0, 0), (0, DP - D)))
    return _sc_call(predicate_ids, entity_ids, ent_pad, pred_pad)
